# baseline (speedup 1.0000x reference)
import jax
import jax.numpy as jnp
from jax.experimental import pallas as pl


def _tail(h_ref, w_ref, b_ref, o_ref):
    o_ref[...] = h_ref[...] @ w_ref[...] + b_ref[...]


def kernel(x, edge_index, edge_attr, en0_w1, en0_b1, en0_w2, en0_b2,
           en1_w1, en1_b1, en1_w2, en1_b2,
           gc_w1, gc_b1, gc_w2, gc_b2, mlp_w, mlp_b):
    src = edge_index[0]
    dst = edge_index[1]
    n = x.shape[0]
    deg = jnp.zeros((n,), jnp.float32).at[dst].add(1.0) + 1.0
    dinv = deg ** -0.5

    def gcn(h, w, b):
        hw = h @ w
        norm = dinv[src] * dinv[dst]
        acc = jnp.zeros_like(hw).at[dst].add(hw[src] * norm[:, None])
        return jax.nn.relu(acc + (dinv * dinv)[:, None] * hw + b)

    h = gcn(x, en0_w1, en0_b1)
    h2 = gcn(h, en0_w2, en0_b2)

    lens = h2.mean(1)
    t = (lens - lens.min()) / (lens.max() - lens.min() + 1e-9)
    C0 = 512
    ln = 1.0 / C0
    i = jnp.arange(C0, dtype=jnp.float32)
    left = i * ln - 0.1 * ln
    right = (i + 1.0) * ln + 0.1 * ln
    S = ((t[:, None] >= left[None, :]) & (t[:, None] <= right[None, :]))
    iota = jnp.arange(C0, dtype=jnp.int32)
    m_first = jnp.min(jnp.where(S, iota[None, :], 4 * C0), axis=1)
    m_last = jnp.max(jnp.where(S, iota[None, :], -1), axis=1)
    m0 = jnp.where(m_first < 4 * C0, m_first, C0)
    m1 = jnp.where(m_last > m_first, m_last, C0)
    Sf = S.astype(jnp.float32)
    size = jnp.clip(Sf.sum(0), 1.0, None)
    xp = (Sf.T @ h2) / size[:, None]
    W = 528
    ci0 = m0[src]; ci1 = m1[src]; cj0 = m0[dst]; cj1 = m1[dst]
    apf = jnp.zeros((W * W,), jnp.float32)
    for a in (ci0, ci1):
        for bb in (cj0, cj1):
            apf = apf.at[a * W + bb].add(edge_attr)
    ap = apf.reshape(W, W)[:C0, :C0]

    abin = (ap != 0).astype(jnp.float32)

    def dense_gcn(h, adj, w, b):
        nn_ = h.shape[0]
        a = adj + jnp.eye(nn_, dtype=h.dtype)
        degd = a.sum(1)
        dinvd = degd ** -0.5
        an = a * dinvd[:, None] * dinvd[None, :]
        return an @ (h @ w) + b

    h = jax.nn.relu(dense_gcn(xp, abin, en1_w1, en1_b1))
    h = jax.nn.relu(dense_gcn(h, abin, en1_w2, en1_b2))

    C1 = 128
    lens = h.mean(1)
    t = (lens - lens.min()) / (lens.max() - lens.min() + 1e-9)
    ln = 1.0 / C1
    i = jnp.arange(C1, dtype=jnp.float32)
    left = i * ln - 0.1 * ln
    right = (i + 1.0) * ln + 0.1 * ln
    S2 = ((t[:, None] >= left[None, :]) & (t[:, None] <= right[None, :])).astype(jnp.float32)
    size2 = jnp.clip(S2.sum(0), 1.0, None)
    xp2 = (S2.T @ h) / size2[:, None]
    ap2 = S2.T @ ap @ S2

    h = jax.nn.relu(dense_gcn(xp2, ap2, gc_w1, gc_b1))
    h = jax.nn.relu(dense_gcn(h, ap2, gc_w2, gc_b2))
    h = h.mean(0)

    out = pl.pallas_call(
        _tail,
        out_shape=jax.ShapeDtypeStruct((10,), jnp.float32),
    )(h, mlp_w, mlp_b)
    return out


# speedup vs baseline: 6.0654x; 6.0654x over previous
import jax
import jax.numpy as jnp
import numpy as np
from jax import lax
from jax.experimental import pallas as pl
from jax.experimental.pallas import tpu as pltpu
from jax.experimental.pallas import tpu_sc as plsc

N = 4096
E = 65536
D = 128
C0 = 512
C1 = 128
NC = 2
NS = 16
NW = 32

_LN0 = 1.0 / C0
_D0 = float(0.1 * (1.0 / C0))
_LN1 = 1.0 / C1
_D1 = float(0.1 * (1.0 / C1))

_f32 = jnp.float32
_i32 = jnp.int32

_SC_PARAMS = pltpu.CompilerParams(needs_layout_passes=False,
                                  use_tc_tiling_on_sc=False)
_TC_PARAMS = pltpu.CompilerParams(vmem_limit_bytes=100 * 1024 * 1024)
_MESH = dict(mesh=plsc.VectorSubcoreMesh(core_axis_name="c",
                                         subcore_axis_name="s"))


def _iota16():
    return lax.iota(_i32, 16)


def _sc_deg(dstr, zflatn, out, dst_v, acc_v, _):
    cid = lax.axis_index("c")
    sid = lax.axis_index("s")
    wid = sid * NC + cid
    pltpu.sync_copy(zflatn, acc_v)
    pltpu.sync_copy(dstr.at[wid], dst_v)
    ones = jnp.full((16,), 1.0, _f32)

    def body(j, _):
        for h in range(8):
            d16 = dst_v[j, pl.ds(h * 16, 16)]
            plsc.addupdate_scatter(acc_v, [d16], ones)
        return 0

    lax.fori_loop(0, 16, body, 0)
    pltpu.sync_copy(acc_v, out.at[wid])


def _deg_call(dstr, zflatn):
    return pl.kernel(
        _sc_deg,
        out_type=jax.ShapeDtypeStruct((NW, N), _f32),
        compiler_params=_SC_PARAMS, **_MESH,
        scratch_types=[
            pltpu.VMEM((16, 128), _i32),
            pltpu.VMEM((N,), _f32),
            pltpu.SemaphoreType.DMA,
        ],
    )(dstr, zflatn)


def _sc_rowscat(gview, srcr, dstr, dinvn, zacc, out,
                src_v, dst_v, gidx_v, rows_v, dinv_v, acc_v, sem):
    cid = lax.axis_index("c")
    sid = lax.axis_index("s")
    wid = sid * NC + cid
    g = lax.shift_right_logical(wid, 3)
    cb = lax.bitwise_and(wid, 7)
    pltpu.sync_copy(zacc, acc_v)
    pltpu.sync_copy(dinvn, dinv_v)
    iota = _iota16()

    def body(j, _):
        pltpu.sync_copy(srcr.at[g, j], src_v)
        pltpu.sync_copy(dstr.at[g, j], dst_v)
        for r in range(8):
            for h in range(8):
                s16 = src_v[r, pl.ds(h * 16, 16)]
                gidx_v[r, pl.ds(h * 16, 16)] = s16 * 8 + cb
        cps = []
        for r in range(8):
            cps.append(pltpu.async_copy(
                gview.at[gidx_v.at[r]], rows_v.at[pl.ds(r * 128, 128)], sem))
        for cp in cps:
            cp.wait()
        for r in range(8):
            for h in range(8):
                s16 = src_v[r, pl.ds(h * 16, 16)]
                d16 = dst_v[r, pl.ds(h * 16, 16)]
                nv = (plsc.load_gather(dinv_v, [s16])
                      * plsc.load_gather(dinv_v, [d16]))
                rbase = r * 128 + h * 16
                for l in range(16):
                    lane = jnp.full((16,), l, _i32)
                    vals = plsc.load_gather(rows_v, [iota + rbase, lane])
                    plsc.addupdate_scatter(acc_v, [d16, lane], vals * nv)
        return 0

    lax.fori_loop(0, 16, body, 0)
    pltpu.sync_copy(acc_v, out.at[g, cb])


def _rowscat_call(gview, srcr, dstr, dinvn, zacc):
    return pl.kernel(
        _sc_rowscat,
        out_type=jax.ShapeDtypeStruct((4, 8, N, 16), _f32),
        compiler_params=_SC_PARAMS, **_MESH,
        scratch_types=[
            pltpu.VMEM((8, 128), _i32),
            pltpu.VMEM((8, 128), _i32),
            pltpu.VMEM((8, 128), _i32),
            pltpu.VMEM((1024, 16), _f32),
            pltpu.VMEM((N,), _f32),
            pltpu.VMEM((N, 16), _f32),
            pltpu.SemaphoreType.DMA,
        ],
    )(gview, srcr, dstr, dinvn, zacc)


def _sc_apscat(m0, m1, srcr, dstr, attrr, zflat, out,
               m0_v, m1_v, src_v, dst_v, attr_v, acc_v, _):
    cid = lax.axis_index("c")
    sid = lax.axis_index("s")
    wid = sid * NC + cid
    g = lax.shift_right_logical(wid, 2)
    r = lax.bitwise_and(wid, 3)
    lo = r * 128
    pltpu.sync_copy(zflat, acc_v)
    pltpu.sync_copy(m0, m0_v)
    pltpu.sync_copy(m1, m1_v)

    def body(j, _):
        pltpu.sync_copy(srcr.at[g, j], src_v)
        pltpu.sync_copy(dstr.at[g, j], dst_v)
        pltpu.sync_copy(attrr.at[g, j], attr_v)
        for q in range(8):
            for h in range(8):
                s16 = src_v[q, pl.ds(h * 16, 16)]
                d16 = dst_v[q, pl.ds(h * 16, 16)]
                att = attr_v[q, pl.ds(h * 16, 16)]
                ci0 = plsc.load_gather(m0_v, [s16])
                ci1 = plsc.load_gather(m1_v, [s16])
                cj0 = plsc.load_gather(m0_v, [d16])
                cj1 = plsc.load_gather(m1_v, [d16])
                for ci, cj in ((ci0, cj0), (ci0, cj1), (ci1, cj0), (ci1, cj1)):
                    cil = ci - lo
                    msk = (ci >= lo) & (ci < lo + 128) & (cj < C0)
                    f = cil * C0 + cj
                    plsc.addupdate_scatter(acc_v, [f], att, mask=msk)
        return 0

    lax.fori_loop(0, 8, body, 0)
    pltpu.sync_copy(acc_v, out.at[g, r])


def _apscat_call(m0, m1, srcr, dstr, attrr, zflat):
    return pl.kernel(
        _sc_apscat,
        out_type=jax.ShapeDtypeStruct((8, 4, 128 * C0), _f32),
        compiler_params=_SC_PARAMS, **_MESH,
        scratch_types=[
            pltpu.VMEM((N,), _i32),
            pltpu.VMEM((N,), _i32),
            pltpu.VMEM((8, 128), _i32),
            pltpu.VMEM((8, 128), _i32),
            pltpu.VMEM((8, 128), _f32),
            pltpu.VMEM((128 * C0,), _f32),
            pltpu.SemaphoreType.DMA,
        ],
    )(m0, m1, srcr, dstr, attrr, zflat)


def _tc_prep(x_ref, w_ref, degp_ref, g_ref, dinv_ref):
    deg = jnp.sum(degp_ref[...], axis=1, keepdims=True) + 1.0
    dinv = lax.rsqrt(deg)
    hw = jnp.dot(x_ref[...], w_ref[...], preferred_element_type=_f32)
    g_ref[...] = hw
    dinv_ref[...] = dinv


def _combine_acc(accp_ref):
    return (accp_ref[0] + accp_ref[1] + accp_ref[2] + accp_ref[3])


def _tc_mid(accp_ref, g_ref, dinv_ref, b_ref, w_ref, o_ref):
    dinv = dinv_ref[...]
    acc = _combine_acc(accp_ref) + (dinv * dinv) * g_ref[...]
    h = jnp.maximum(acc + b_ref[...], 0.0)
    o_ref[...] = jnp.dot(h, w_ref[...], preferred_element_type=_f32)


def _tc_h2(accp_ref, g_ref, dinv_ref, b_ref, h2_ref):
    dinv = dinv_ref[...]
    acc = _combine_acc(accp_ref) + (dinv * dinv) * g_ref[...]
    h2_ref[...] = jnp.maximum(acc + b_ref[...], 0.0)


def _member(t, c):
    cf = c.astype(_f32)
    return ((c >= 0) & (c <= C0 - 1)
            & (t >= cf * _LN0 - _D0) & (t <= (cf + 1.0) * _LN0 + _D0))


def _tc_pool0(h2_ref, xp_ref, m0_ref, m1_ref):
    h2 = h2_ref[...]
    lens = jnp.mean(h2, axis=1, keepdims=True)
    tmin = jnp.min(lens)
    tmax = jnp.max(lens)
    t = (lens - tmin) / (tmax - tmin + 1e-9)
    kf = (t * float(C0)).astype(_i32)
    ca, cb_, cc = kf - 1, kf, kf + 1
    ma, mb, mc = _member(t, ca), _member(t, cb_), _member(t, cc)
    m0 = jnp.where(ma, ca, jnp.where(mb, cb_, jnp.where(mc, cc, C0)))
    mlast = jnp.where(mc, cc, jnp.where(mb, cb_, jnp.where(ma, ca, C0)))
    m0_ref[...] = m0
    m1_ref[...] = jnp.where((mlast > m0) & (mlast < C0), mlast, C0)
    iotaf = lax.broadcasted_iota(_i32, (1, C0), 1).astype(_f32)
    left = iotaf * _LN0 - _D0
    right = (iotaf + 1.0) * _LN0 + _D0
    Sf = ((t >= left) & (t <= right)).astype(_f32)
    ones_col = jnp.full((N, 1), 1.0, _f32)
    size = lax.dot_general(Sf, ones_col, (((0,), (0,)), ((), ())),
                           preferred_element_type=_f32)
    xpr = lax.dot_general(Sf, h2, (((0,), (0,)), ((), ())),
                          preferred_element_type=_f32)
    xp_ref[...] = xpr / jnp.maximum(size, 1.0)


def _dense_gcn(h, a_noloop, w, b_row):
    n_ = a_noloop.shape[0]
    ii = lax.broadcasted_iota(_i32, (n_, n_), 0)
    jj = lax.broadcasted_iota(_i32, (n_, n_), 1)
    aa = a_noloop + jnp.where(ii == jj, 1.0, 0.0).astype(_f32)
    degd = jnp.sum(aa, axis=1, keepdims=True)
    dinvd = lax.rsqrt(degd)
    hw = jnp.dot(h, w, preferred_element_type=_f32)
    return dinvd * jnp.dot(aa, dinvd * hw, preferred_element_type=_f32) + b_row


def _tc_tail(app_ref, xp_ref, w1_ref, b1_ref, w2_ref, b2_ref,
             g1_ref, gb1_ref, g2_ref, gb2_ref, mw_ref, mb_ref, o_ref):
    ap = app_ref[0]
    for g in range(1, 8):
        ap = ap + app_ref[g]
    abin = (ap != 0.0).astype(_f32)
    h = jnp.maximum(_dense_gcn(xp_ref[...], abin, w1_ref[...], b1_ref[...]), 0.0)
    h = jnp.maximum(_dense_gcn(h, abin, w2_ref[...], b2_ref[...]), 0.0)
    lens = jnp.mean(h, axis=1, keepdims=True)
    tmin = jnp.min(lens)
    tmax = jnp.max(lens)
    t = (lens - tmin) / (tmax - tmin + 1e-9)
    iotaf = lax.broadcasted_iota(_i32, (1, C1), 1).astype(_f32)
    left = iotaf * _LN1 - _D1
    right = (iotaf + 1.0) * _LN1 + _D1
    S2 = ((t >= left) & (t <= right)).astype(_f32)
    ones_col = jnp.full((C0, 1), 1.0, _f32)
    size2 = lax.dot_general(S2, ones_col, (((0,), (0,)), ((), ())),
                            preferred_element_type=_f32)
    xp2 = lax.dot_general(S2, h, (((0,), (0,)), ((), ())),
                          preferred_element_type=_f32) / jnp.maximum(size2, 1.0)
    ap2l = lax.dot_general(S2, ap, (((0,), (0,)), ((), ())),
                           preferred_element_type=_f32)
    ap2 = jnp.dot(ap2l, S2, preferred_element_type=_f32)
    h = jnp.maximum(_dense_gcn(xp2, ap2, g1_ref[...], gb1_ref[...]), 0.0)
    h = jnp.maximum(_dense_gcn(h, ap2, g2_ref[...], gb2_ref[...]), 0.0)
    hm = jnp.mean(h, axis=0, keepdims=True)
    o_ref[...] = jnp.dot(hm, mw_ref[...], preferred_element_type=_f32) + mb_ref[...]


def kernel(x, edge_index, edge_attr, en0_w1, en0_b1, en0_w2, en0_b2,
           en1_w1, en1_b1, en1_w2, en1_b2,
           gc_w1, gc_b1, gc_w2, gc_b2, mlp_w, mlp_b):
    src = edge_index[0]
    dst = edge_index[1]
    dstr_d = dst.reshape(NW, 16, 128)
    srcr_a = src.reshape(4, 16, 8, 128)
    dstr_a = dst.reshape(4, 16, 8, 128)
    srcr_b = src.reshape(8, 8, 8, 128)
    dstr_b = dst.reshape(8, 8, 8, 128)
    attrr_b = edge_attr.reshape(8, 8, 8, 128)
    zacc = jnp.zeros((N, 16), _f32)
    zflat = jnp.zeros((128 * C0,), _f32)
    zflatn = jnp.zeros((N,), _f32)

    degp = _deg_call(dstr_d, zflatn).T

    g1, dinv = pl.pallas_call(
        _tc_prep,
        compiler_params=_TC_PARAMS,
        out_shape=(jax.ShapeDtypeStruct((N, D), _f32),
                   jax.ShapeDtypeStruct((N, 1), _f32)),
    )(x, en0_w1, degp)

    dinvn = dinv.reshape(N)
    acc1 = _rowscat_call(g1.reshape(N * 8, 16), srcr_a, dstr_a, dinvn, zacc)
    acc1 = acc1.transpose(0, 2, 1, 3).reshape(4, N, D)

    g2 = pl.pallas_call(
        _tc_mid,
        compiler_params=_TC_PARAMS,
        out_shape=jax.ShapeDtypeStruct((N, D), _f32),
    )(acc1, g1, dinv, en0_b1.reshape(1, D), en0_w2)

    acc2 = _rowscat_call(g2.reshape(N * 8, 16), srcr_a, dstr_a, dinvn, zacc)
    acc2 = acc2.transpose(0, 2, 1, 3).reshape(4, N, D)

    h2 = pl.pallas_call(
        _tc_h2,
        compiler_params=_TC_PARAMS,
        out_shape=jax.ShapeDtypeStruct((N, D), _f32),
    )(acc2, g2, dinv, en0_b2.reshape(1, D))

    xp, m0, m1 = pl.pallas_call(
        _tc_pool0,
        compiler_params=_TC_PARAMS,
        out_shape=(jax.ShapeDtypeStruct((C0, D), _f32),
                   jax.ShapeDtypeStruct((N, 1), _i32),
                   jax.ShapeDtypeStruct((N, 1), _i32)),
    )(h2)

    app = _apscat_call(m0.reshape(N), m1.reshape(N), srcr_b, dstr_b,
                       attrr_b, zflat)

    out = pl.pallas_call(
        _tc_tail,
        compiler_params=_TC_PARAMS,
        out_shape=jax.ShapeDtypeStruct((1, 10), _f32),
    )(app.reshape(8, C0, C0), xp,
      en1_w1, en1_b1.reshape(1, D), en1_w2, en1_b2.reshape(1, D),
      gc_w1, gc_b1.reshape(1, D), gc_w2, gc_b2.reshape(1, D),
      mlp_w, mlp_b.reshape(1, 10))
    return out.reshape(10)
